# baseline (speedup 1.0000x reference)
import functools

import jax
import jax.numpy as jnp
from jax import lax
from jax.experimental import pallas as pl
from jax.experimental.pallas import tpu as pltpu
from jax.experimental.pallas import tpu_sc as plsc

BATCH = 16384
D = 16
NC = 2
NS = 16
NW = NC * NS
PER_W = BATCH // NW
CHUNK = 128
NCH = PER_W // CHUNK
ROWW = 128
NTAB = 4
NVOC = 1000000
VROWS = (NVOC * D) // ROWW
BPC = CHUNK // D
SLAB = 128
SROWS = SLAB // 8
NFULL = NVOC // SLAB
TAILB = NFULL * SLAB
NPAIR = ((NFULL + NW - 1) // NW + 1) // 2
MESH = plsc.VectorSubcoreMesh(core_axis_name="c", subcore_axis_name="s")
PARAMS = pltpu.CompilerParams(needs_layout_passes=False)


def _relayout_body(t0, t1, t2, t3, aux_hbm, out_hbm, slab_v, blk_v, sem_r0,
                   sem_r1, sem_w0, sem_w1):
    sem_r = (sem_r0, sem_r1)
    sem_w = (sem_w0, sem_w1)
    cid = lax.axis_index("c")
    sid = lax.axis_index("s")
    wid = sid * NC + cid
    tbls = (t0, t1, t2, t3)
    iota = lax.iota(jnp.int32, D)

    @pl.when(wid == 0)
    def _patch_tail():
        for t in range(NTAB):
            pltpu.sync_copy(aux_hbm.at[t],
                            out_hbm.at[t, pl.ds(TAILB // 8, 8), :])

    n_w = (NFULL - wid + NW - 1) // NW

    def read(s, i):
        start = (wid + i * NW) * SLAB
        for t in range(NTAB):
            pltpu.async_copy(tbls[t].at[:, pl.ds(start, SLAB)],
                             slab_v.at[s, t], sem_r[s])

    def wait_read(s, i):
        start = (wid + i * NW) * SLAB
        for t in range(NTAB):
            pltpu.make_async_copy(tbls[t].at[:, pl.ds(start, SLAB)],
                                  slab_v.at[s, t], sem_r[s]).wait()

    def write(s, i):
        rowb = (wid + i * NW) * SROWS
        for t in range(NTAB):
            pltpu.async_copy(blk_v.at[s, t],
                             out_hbm.at[t, pl.ds(rowb, SROWS), :], sem_w[s])

    def wait_write(s, i):
        rowb = (wid + i * NW) * SROWS
        for t in range(NTAB):
            pltpu.make_async_copy(blk_v.at[s, t],
                                  out_hbm.at[t, pl.ds(rowb, SROWS), :],
                                  sem_w[s]).wait()

    def shuffle(s):
        for t in range(NTAB):
            @plsc.parallel_loop(0, SROWS, unroll=8)
            def _row(m):
                for v in range(8):
                    col = iota * 0 + (8 * m + v)
                    blk_v[s, t, m, pl.ds(16 * v, D)] = plsc.load_gather(
                        slab_v.at[s, t], [iota, col])

    read(0, 0)

    def pair(j, _):
        for s in range(2):
            i = 2 * j + s

            @pl.when(i < n_w)
            def _do():
                @pl.when(i + 1 < n_w)
                def _pref():
                    read(1 - s, i + 1)

                @pl.when(i >= 2)
                def _wb():
                    wait_write(s, i - 2)
                wait_read(s, i)
                shuffle(s)
                write(s, i)
        return _

    lax.fori_loop(0, NPAIR, pair, 0)
    wait_write(0, 0)
    wait_write(1, 0)


@functools.partial(
    pl.kernel,
    out_type=jax.ShapeDtypeStruct((NTAB, VROWS, ROWW), jnp.float32),
    mesh=MESH,
    compiler_params=PARAMS,
    scratch_types=[
        pltpu.VMEM((2, NTAB, D, SLAB), jnp.float32),
        pltpu.VMEM((2, NTAB, SROWS, ROWW), jnp.float32),
        pltpu.SemaphoreType.DMA,
        pltpu.SemaphoreType.DMA,
        pltpu.SemaphoreType.DMA,
        pltpu.SemaphoreType.DMA,
    ],
)
def _relayout_sc(t0, t1, t2, t3, aux_hbm, out_hbm, *scratch):
    _relayout_body(t0, t1, t2, t3, aux_hbm, out_hbm, *scratch)


def _gather_body(gids_hbm, loffs_hbm, tab_hbm, w_hbm, c_hbm, out_hbm,
                 gids_v, loffs_v, g0, g1, g2, g3, w_v, c_v, out_v, sem):
    cid = lax.axis_index("c")
    sid = lax.axis_index("s")
    wid = sid * NC + cid
    gbufs = (g0, g1, g2, g3)

    pltpu.sync_copy(gids_hbm.at[wid, 0], gids_v)
    pltpu.sync_copy(loffs_hbm.at[wid, 0], loffs_v)
    pltpu.sync_copy(w_hbm, w_v)
    pltpu.sync_copy(c_hbm, c_v)

    iota = lax.iota(jnp.int32, D)
    c_splat = c_v[pl.ds(0, D)]
    wmf_rows = [w_v[0, k, pl.ds(0, D)] for k in range(D)]
    wa_rows = [w_v[1, k, pl.ds(0, D)] for k in range(D)]
    wb_rows = [w_v[2, k, pl.ds(0, D)] for k in range(D)]

    for r in range(NCH):
        copies = []
        for t in range(NTAB):
            row = r if t % 2 == 0 else NCH + r
            copies.append(pltpu.async_copy(
                tab_hbm.at[t].at[gids_v.at[row]], gbufs[t], sem))
        for cp in copies:
            cp.wait()

        def blk(b, _):
            rows = b * D + iota
            ucols = loffs_v[r, pl.ds(b * D, D)]
            icols = loffs_v[NCH + r, pl.ds(b * D, D)]
            acc = c_splat
            for k in range(D):
                u1 = plsc.load_gather(g0, [rows, ucols + k])
                i1 = plsc.load_gather(g1, [rows, icols + k])
                u2 = plsc.load_gather(g2, [rows, ucols + k])
                i2 = plsc.load_gather(g3, [rows, icols + k])
                acc = (acc + u1 * i1 * wmf_rows[k]
                       + u2 * wa_rows[k] + i2 * wb_rows[k])
            out_v[pl.ds(r * CHUNK + b * D, D)] = 1.0 / (1.0 + jnp.exp(-acc))
            return _

        lax.fori_loop(0, BPC, blk, 0)

    pltpu.sync_copy(out_v, out_hbm.at[pl.ds(wid * PER_W, PER_W)])


@functools.partial(
    pl.kernel,
    out_type=jax.ShapeDtypeStruct((BATCH,), jnp.float32),
    mesh=MESH,
    compiler_params=PARAMS,
    scratch_types=[
        pltpu.VMEM((2 * NCH, CHUNK), jnp.int32),
        pltpu.VMEM((2 * NCH, CHUNK), jnp.int32),
        pltpu.VMEM((CHUNK, ROWW), jnp.float32),
        pltpu.VMEM((CHUNK, ROWW), jnp.float32),
        pltpu.VMEM((CHUNK, ROWW), jnp.float32),
        pltpu.VMEM((CHUNK, ROWW), jnp.float32),
        pltpu.VMEM((3, D, 128), jnp.float32),
        pltpu.VMEM((128,), jnp.float32),
        pltpu.VMEM((PER_W,), jnp.float32),
        pltpu.SemaphoreType.DMA,
    ],
)
def _neumf_sc(gids_hbm, loffs_hbm, tab_hbm, w_hbm, c_hbm, out_hbm, *scratch):
    _gather_body(gids_hbm, loffs_hbm, tab_hbm, w_hbm, c_hbm, out_hbm, *scratch)


def kernel(X, user_mf, item_mf, user_mlp, item_mlp, W_mlp, b_mlp, W_pred, b_pred):
    tabs = (user_mf, item_mf, user_mlp, item_mlp)

    Xi = X.astype(jnp.int32)
    gid = Xi >> 3
    loff = (Xi & 7) * D
    pack = lambda A: jnp.concatenate(
        [A[:, 0].reshape(NW, NCH, CHUNK), A[:, 1].reshape(NW, NCH, CHUNK)],
        axis=1).reshape(NW, 1, 2 * NCH, CHUNK)
    gids = pack(gid)
    loffs = pack(loff)

    aux = jnp.stack([t[TAILB:].reshape(8, ROWW) for t in tabs])

    h = W_pred[D:, 0]
    a = W_mlp[:D, :] @ h
    b = W_mlp[D:, :] @ h
    c = b_mlp @ h + b_pred[0]
    wmf = W_pred[:D, 0]
    w_vecs = jnp.stack([wmf, a, b]).astype(jnp.float32)
    w_rows = jnp.tile(w_vecs[:, :, None], (1, 1, 128))
    c_vec = jnp.full((128,), c, jnp.float32)

    rows = _relayout_sc(user_mf.T, item_mf.T, user_mlp.T, item_mlp.T, aux)
    out = _neumf_sc(gids, loffs, rows, w_rows, c_vec)
    return out.reshape(BATCH, 1)


# speedup vs baseline: 3.2628x; 3.2628x over previous
import functools

import jax
import jax.numpy as jnp
from jax import lax
from jax.experimental import pallas as pl
from jax.experimental.pallas import tpu as pltpu
from jax.experimental.pallas import tpu_sc as plsc

BATCH = 16384
D = 16
NC = 2
NS = 16
NW = NC * NS
PER_W = BATCH // NW
CHUNK = 128
NCH = PER_W // CHUNK
ROWW = 128
NTAB = 4
NVOC = 1000000
VROWS = (NVOC * D) // ROWW
BPC = CHUNK // D
SLAB = 128
SROWS = SLAB // 8
NFULL = NVOC // SLAB
TAILB = NFULL * SLAB
NPAIR = ((NFULL + NW - 1) // NW + 1) // 2
MESH = plsc.VectorSubcoreMesh(core_axis_name="c", subcore_axis_name="s")
PARAMS = pltpu.CompilerParams(needs_layout_passes=False)


def _relayout_body(t0, t1, t2, t3, aux_hbm, out_hbm, slab_v, blk_v, sem_r0,
                   sem_r1, sem_w0, sem_w1):
    sem_r = (sem_r0, sem_r1)
    sem_w = (sem_w0, sem_w1)
    cid = lax.axis_index("c")
    sid = lax.axis_index("s")
    wid = sid * NC + cid
    tbls = (t0, t1, t2, t3)
    iota = lax.iota(jnp.int32, D)

    @pl.when(wid == 0)
    def _patch_tail():
        for t in range(NTAB):
            pltpu.sync_copy(aux_hbm.at[t],
                            out_hbm.at[t, pl.ds(TAILB // 8, 8), :])

    n_w = (NFULL - wid + NW - 1) // NW

    def read(s, i):
        start = (wid + i * NW) * SLAB
        for t in range(NTAB):
            pltpu.async_copy(tbls[t].at[:, pl.ds(start, SLAB)],
                             slab_v.at[s, t], sem_r[s])

    def wait_read(s, i):
        start = (wid + i * NW) * SLAB
        for t in range(NTAB):
            pltpu.make_async_copy(tbls[t].at[:, pl.ds(start, SLAB)],
                                  slab_v.at[s, t], sem_r[s]).wait()

    def write(s, i):
        rowb = (wid + i * NW) * SROWS
        for t in range(NTAB):
            pltpu.async_copy(blk_v.at[s, t],
                             out_hbm.at[t, pl.ds(rowb, SROWS), :], sem_w[s])

    def wait_write(s, i):
        rowb = (wid + i * NW) * SROWS
        for t in range(NTAB):
            pltpu.make_async_copy(blk_v.at[s, t],
                                  out_hbm.at[t, pl.ds(rowb, SROWS), :],
                                  sem_w[s]).wait()

    iota_d8 = iota // 8
    iota_m8 = iota % 8

    def shuffle(s):
        for t in range(NTAB):
            @plsc.parallel_loop(0, SROWS, unroll=4)
            def _row(m):
                for v in range(8):
                    rowv = iota_d8 + 2 * v
                    colv = iota_m8 + 8 * m
                    blk_v[s, t, m, pl.ds(16 * v, D)] = plsc.load_gather(
                        slab_v.at[s, t], [rowv, colv])

    read(0, 0)

    def pair(j, _):
        for s in range(2):
            i = 2 * j + s

            @pl.when(i < n_w)
            def _do():
                @pl.when(i + 1 < n_w)
                def _pref():
                    read(1 - s, i + 1)

                @pl.when(i >= 2)
                def _wb():
                    wait_write(s, i - 2)
                wait_read(s, i)
                shuffle(s)
                write(s, i)
        return _

    lax.fori_loop(0, NPAIR, pair, 0)
    wait_write(0, 0)
    wait_write(1, 0)


@functools.partial(
    pl.kernel,
    out_type=jax.ShapeDtypeStruct((NTAB, VROWS, ROWW), jnp.float32),
    mesh=MESH,
    compiler_params=PARAMS,
    scratch_types=[
        pltpu.VMEM((2, NTAB, D, SLAB), jnp.float32),
        pltpu.VMEM((2, NTAB, SROWS, ROWW), jnp.float32),
        pltpu.SemaphoreType.DMA,
        pltpu.SemaphoreType.DMA,
        pltpu.SemaphoreType.DMA,
        pltpu.SemaphoreType.DMA,
    ],
)
def _relayout_sc(t0, t1, t2, t3, aux_hbm, out_hbm, *scratch):
    _relayout_body(t0, t1, t2, t3, aux_hbm, out_hbm, *scratch)


def _gather_body(gids_hbm, loffs_hbm, tab_hbm, w_hbm, c_hbm, out_hbm,
                 gids_v, loffs_v, g0, g1, g2, g3, w_v, c_v, out_v, sem):
    cid = lax.axis_index("c")
    sid = lax.axis_index("s")
    wid = sid * NC + cid
    gbufs = (g0, g1, g2, g3)

    pltpu.sync_copy(gids_hbm.at[wid, 0], gids_v)
    pltpu.sync_copy(loffs_hbm.at[wid, 0], loffs_v)
    pltpu.sync_copy(w_hbm, w_v)
    pltpu.sync_copy(c_hbm, c_v)

    iota = lax.iota(jnp.int32, D)
    c_splat = c_v[pl.ds(0, D)]
    wmf_rows = [w_v[0, k, pl.ds(0, D)] for k in range(D)]
    wa_rows = [w_v[1, k, pl.ds(0, D)] for k in range(D)]
    wb_rows = [w_v[2, k, pl.ds(0, D)] for k in range(D)]

    for r in range(NCH):
        copies = []
        for t in range(NTAB):
            row = r if t % 2 == 0 else NCH + r
            copies.append(pltpu.async_copy(
                tab_hbm.at[t].at[gids_v.at[row]], gbufs[t], sem))
        for cp in copies:
            cp.wait()

        def blk(b, _):
            rows = b * D + iota
            ucols = loffs_v[r, pl.ds(b * D, D)]
            icols = loffs_v[NCH + r, pl.ds(b * D, D)]
            acc = c_splat
            for k in range(D):
                u1 = plsc.load_gather(g0, [rows, ucols + 8 * k])
                i1 = plsc.load_gather(g1, [rows, icols + 8 * k])
                u2 = plsc.load_gather(g2, [rows, ucols + 8 * k])
                i2 = plsc.load_gather(g3, [rows, icols + 8 * k])
                acc = (acc + u1 * i1 * wmf_rows[k]
                       + u2 * wa_rows[k] + i2 * wb_rows[k])
            out_v[pl.ds(r * CHUNK + b * D, D)] = 1.0 / (1.0 + jnp.exp(-acc))
            return _

        lax.fori_loop(0, BPC, blk, 0)

    pltpu.sync_copy(out_v, out_hbm.at[pl.ds(wid * PER_W, PER_W)])


@functools.partial(
    pl.kernel,
    out_type=jax.ShapeDtypeStruct((BATCH,), jnp.float32),
    mesh=MESH,
    compiler_params=PARAMS,
    scratch_types=[
        pltpu.VMEM((2 * NCH, CHUNK), jnp.int32),
        pltpu.VMEM((2 * NCH, CHUNK), jnp.int32),
        pltpu.VMEM((CHUNK, ROWW), jnp.float32),
        pltpu.VMEM((CHUNK, ROWW), jnp.float32),
        pltpu.VMEM((CHUNK, ROWW), jnp.float32),
        pltpu.VMEM((CHUNK, ROWW), jnp.float32),
        pltpu.VMEM((3, D, 128), jnp.float32),
        pltpu.VMEM((128,), jnp.float32),
        pltpu.VMEM((PER_W,), jnp.float32),
        pltpu.SemaphoreType.DMA,
    ],
)
def _neumf_sc(gids_hbm, loffs_hbm, tab_hbm, w_hbm, c_hbm, out_hbm, *scratch):
    _gather_body(gids_hbm, loffs_hbm, tab_hbm, w_hbm, c_hbm, out_hbm, *scratch)


def kernel(X, user_mf, item_mf, user_mlp, item_mlp, W_mlp, b_mlp, W_pred, b_pred):
    tabs = (user_mf, item_mf, user_mlp, item_mlp)

    Xi = X.astype(jnp.int32)
    gid = Xi >> 3
    loff = Xi & 7
    pack = lambda A: jnp.concatenate(
        [A[:, 0].reshape(NW, NCH, CHUNK), A[:, 1].reshape(NW, NCH, CHUNK)],
        axis=1).reshape(NW, 1, 2 * NCH, CHUNK)
    gids = pack(gid)
    loffs = pack(loff)

    aux = jnp.stack([
        t[TAILB:].reshape(8, 8, D).transpose(0, 2, 1).reshape(8, ROWW)
        for t in tabs])

    h = W_pred[D:, 0]
    a = W_mlp[:D, :] @ h
    b = W_mlp[D:, :] @ h
    c = b_mlp @ h + b_pred[0]
    wmf = W_pred[:D, 0]
    w_vecs = jnp.stack([wmf, a, b]).astype(jnp.float32)
    w_rows = jnp.tile(w_vecs[:, :, None], (1, 1, 128))
    c_vec = jnp.full((128,), c, jnp.float32)

    rows = _relayout_sc(user_mf.T, item_mf.T, user_mlp.T, item_mlp.T, aux)
    out = _neumf_sc(gids, loffs, rows, w_rows, c_vec)
    return out.reshape(BATCH, 1)


# speedup vs baseline: 3.9536x; 1.2117x over previous
import functools

import jax
import jax.numpy as jnp
from jax import lax
from jax.experimental import pallas as pl
from jax.experimental.pallas import tpu as pltpu
from jax.experimental.pallas import tpu_sc as plsc

BATCH = 16384
D = 16
NC = 2
NS = 16
NW = NC * NS
PER_W = BATCH // NW
CHUNK = 128
NCH = PER_W // CHUNK
ROWW = 128
NTAB = 4
NVOC = 1000000
VROWS = (NVOC * D) // ROWW
BPC = CHUNK // D
SLAB = 256
SROWS = SLAB // 8
NFULL = NVOC // SLAB
TAILB = NFULL * SLAB
NPAIR = ((NFULL + NW - 1) // NW + 1) // 2
MESH = plsc.VectorSubcoreMesh(core_axis_name="c", subcore_axis_name="s")
PARAMS = pltpu.CompilerParams(needs_layout_passes=False)


def _relayout_body(t0, t1, t2, t3, aux_hbm, out_hbm, slab_v, blk_v, sem_r0,
                   sem_r1, sem_w0, sem_w1):
    sem_r = (sem_r0, sem_r1)
    sem_w = (sem_w0, sem_w1)
    cid = lax.axis_index("c")
    sid = lax.axis_index("s")
    wid = sid * NC + cid
    tbls = (t0, t1, t2, t3)
    iota = lax.iota(jnp.int32, D)

    @pl.when(wid == 0)
    def _patch_tail():
        for t in range(NTAB):
            pltpu.sync_copy(aux_hbm.at[t],
                            out_hbm.at[t, pl.ds(TAILB // 8, 8), :])

    n_w = (NFULL - wid + NW - 1) // NW

    def read(s, i):
        start = (wid + i * NW) * SLAB
        for t in range(NTAB):
            pltpu.async_copy(tbls[t].at[:, pl.ds(start, SLAB)],
                             slab_v.at[s, t], sem_r[s])

    def wait_read(s, i):
        start = (wid + i * NW) * SLAB
        for t in range(NTAB):
            pltpu.make_async_copy(tbls[t].at[:, pl.ds(start, SLAB)],
                                  slab_v.at[s, t], sem_r[s]).wait()

    def write(s, i):
        rowb = (wid + i * NW) * SROWS
        for t in range(NTAB):
            pltpu.async_copy(blk_v.at[s, t],
                             out_hbm.at[t, pl.ds(rowb, SROWS), :], sem_w[s])

    def wait_write(s, i):
        rowb = (wid + i * NW) * SROWS
        for t in range(NTAB):
            pltpu.make_async_copy(blk_v.at[s, t],
                                  out_hbm.at[t, pl.ds(rowb, SROWS), :],
                                  sem_w[s]).wait()

    iota_d8 = iota // 8
    iota_m8 = iota % 8

    def shuffle(s):
        for t in range(NTAB):
            @plsc.parallel_loop(0, SROWS, unroll=4)
            def _row(m):
                for v in range(8):
                    rowv = iota_d8 + 2 * v
                    colv = iota_m8 + 8 * m
                    blk_v[s, t, m, pl.ds(16 * v, D)] = plsc.load_gather(
                        slab_v.at[s, t], [rowv, colv])

    read(0, 0)

    def pair(j, _):
        for s in range(2):
            i = 2 * j + s

            @pl.when(i < n_w)
            def _do():
                @pl.when(i + 1 < n_w)
                def _pref():
                    read(1 - s, i + 1)

                @pl.when(i >= 2)
                def _wb():
                    wait_write(s, i - 2)
                wait_read(s, i)
                shuffle(s)
                write(s, i)
        return _

    lax.fori_loop(0, NPAIR, pair, 0)
    wait_write(0, 0)
    wait_write(1, 0)


@functools.partial(
    pl.kernel,
    out_type=jax.ShapeDtypeStruct((NTAB, VROWS, ROWW), jnp.float32),
    mesh=MESH,
    compiler_params=PARAMS,
    scratch_types=[
        pltpu.VMEM((2, NTAB, D, SLAB), jnp.float32),
        pltpu.VMEM((2, NTAB, SROWS, ROWW), jnp.float32),
        pltpu.SemaphoreType.DMA,
        pltpu.SemaphoreType.DMA,
        pltpu.SemaphoreType.DMA,
        pltpu.SemaphoreType.DMA,
    ],
)
def _relayout_sc(t0, t1, t2, t3, aux_hbm, out_hbm, *scratch):
    _relayout_body(t0, t1, t2, t3, aux_hbm, out_hbm, *scratch)


def _gather_body(gids_hbm, loffs_hbm, tab_hbm, w_hbm, c_hbm, out_hbm,
                 gids_v, loffs_v, g0, g1, g2, g3, w_v, c_v, out_v, sem):
    cid = lax.axis_index("c")
    sid = lax.axis_index("s")
    wid = sid * NC + cid
    gbufs = (g0, g1, g2, g3)

    pltpu.sync_copy(gids_hbm.at[wid, 0], gids_v)
    pltpu.sync_copy(loffs_hbm.at[wid, 0], loffs_v)
    pltpu.sync_copy(w_hbm, w_v)
    pltpu.sync_copy(c_hbm, c_v)

    iota = lax.iota(jnp.int32, D)
    c_splat = c_v[pl.ds(0, D)]
    wmf_rows = [w_v[0, k, pl.ds(0, D)] for k in range(D)]
    wa_rows = [w_v[1, k, pl.ds(0, D)] for k in range(D)]
    wb_rows = [w_v[2, k, pl.ds(0, D)] for k in range(D)]

    for r in range(NCH):
        copies = []
        for t in range(NTAB):
            row = r if t % 2 == 0 else NCH + r
            copies.append(pltpu.async_copy(
                tab_hbm.at[t].at[gids_v.at[row]], gbufs[t], sem))
        for cp in copies:
            cp.wait()

        def blk(b, _):
            rows = b * D + iota
            ucols = loffs_v[r, pl.ds(b * D, D)]
            icols = loffs_v[NCH + r, pl.ds(b * D, D)]
            acc = c_splat
            for k in range(D):
                u1 = plsc.load_gather(g0, [rows, ucols + 8 * k])
                i1 = plsc.load_gather(g1, [rows, icols + 8 * k])
                u2 = plsc.load_gather(g2, [rows, ucols + 8 * k])
                i2 = plsc.load_gather(g3, [rows, icols + 8 * k])
                acc = (acc + u1 * i1 * wmf_rows[k]
                       + u2 * wa_rows[k] + i2 * wb_rows[k])
            out_v[pl.ds(r * CHUNK + b * D, D)] = 1.0 / (1.0 + jnp.exp(-acc))
            return _

        lax.fori_loop(0, BPC, blk, 0)

    pltpu.sync_copy(out_v, out_hbm.at[pl.ds(wid * PER_W, PER_W)])


@functools.partial(
    pl.kernel,
    out_type=jax.ShapeDtypeStruct((BATCH,), jnp.float32),
    mesh=MESH,
    compiler_params=PARAMS,
    scratch_types=[
        pltpu.VMEM((2 * NCH, CHUNK), jnp.int32),
        pltpu.VMEM((2 * NCH, CHUNK), jnp.int32),
        pltpu.VMEM((CHUNK, ROWW), jnp.float32),
        pltpu.VMEM((CHUNK, ROWW), jnp.float32),
        pltpu.VMEM((CHUNK, ROWW), jnp.float32),
        pltpu.VMEM((CHUNK, ROWW), jnp.float32),
        pltpu.VMEM((3, D, 128), jnp.float32),
        pltpu.VMEM((128,), jnp.float32),
        pltpu.VMEM((PER_W,), jnp.float32),
        pltpu.SemaphoreType.DMA,
    ],
)
def _neumf_sc(gids_hbm, loffs_hbm, tab_hbm, w_hbm, c_hbm, out_hbm, *scratch):
    _gather_body(gids_hbm, loffs_hbm, tab_hbm, w_hbm, c_hbm, out_hbm, *scratch)


def kernel(X, user_mf, item_mf, user_mlp, item_mlp, W_mlp, b_mlp, W_pred, b_pred):
    tabs = (user_mf, item_mf, user_mlp, item_mlp)

    Xi = X.astype(jnp.int32)
    gid = Xi >> 3
    loff = Xi & 7
    pack = lambda A: jnp.concatenate(
        [A[:, 0].reshape(NW, NCH, CHUNK), A[:, 1].reshape(NW, NCH, CHUNK)],
        axis=1).reshape(NW, 1, 2 * NCH, CHUNK)
    gids = pack(gid)
    loffs = pack(loff)

    aux = jnp.stack([
        t[TAILB:].reshape(8, 8, D).transpose(0, 2, 1).reshape(8, ROWW)
        for t in tabs])

    h = W_pred[D:, 0]
    a = W_mlp[:D, :] @ h
    b = W_mlp[D:, :] @ h
    c = b_mlp @ h + b_pred[0]
    wmf = W_pred[:D, 0]
    w_vecs = jnp.stack([wmf, a, b]).astype(jnp.float32)
    w_rows = jnp.tile(w_vecs[:, :, None], (1, 1, 128))
    c_vec = jnp.full((128,), c, jnp.float32)

    rows = _relayout_sc(user_mf.T, item_mf.T, user_mlp.T, item_mlp.T, aux)
    out = _neumf_sc(gids, loffs, rows, w_rows, c_vec)
    return out.reshape(BATCH, 1)
